# baseline (speedup 1.0000x reference)
import functools

import jax
import jax.numpy as jnp
from jax import lax
from jax.experimental import pallas as pl
from jax.experimental.pallas import tpu as pltpu
from jax.experimental.pallas import tpu_sc as plsc

DIM = 64
NC = 2
NS = 16
NW = NC * NS
NB = 16


def _make_gather(batch, hist):
    bpw = batch // NW
    npairs = bpw // (2 * NB)

    mesh = plsc.VectorSubcoreMesh(core_axis_name="c", subcore_axis_name="s")

    @functools.partial(
        pl.kernel,
        mesh=mesh,
        out_type=jax.ShapeDtypeStruct((batch, hist, DIM), jnp.float32),
        scratch_types=[
            pltpu.VMEM((bpw, hist), jnp.int32),
            pltpu.VMEM((NB, hist, DIM), jnp.float32),
            pltpu.VMEM((NB, hist, DIM), jnp.float32),
            pltpu.SemaphoreType.DMA,
            pltpu.SemaphoreType.DMA,
            pltpu.SemaphoreType.DMA,
            pltpu.SemaphoreType.DMA,
        ],
        compiler_params=pltpu.CompilerParams(use_tc_tiling_on_sc=False),
    )
    def gather(x_hbm, w_hbm, out_hbm, idx_v, buf0, buf1, g0, g1, s0, s1):
        wid = lax.axis_index("s") * NC + lax.axis_index("c")
        base = wid * bpw

        pltpu.sync_copy(x_hbm.at[pl.ds(base, bpw)], idx_v)

        def fire(k, buf, sem):
            return [
                pltpu.async_copy(
                    w_hbm.at[idx_v.at[k * NB + e]],
                    buf.at[e],
                    sem,
                )
                for e in range(NB)
            ]

        def wait_store(buf, sem):
            pltpu.make_async_copy(buf, out_hbm.at[pl.ds(base, NB)], sem).wait()

        def pair(p, carry):
            ka = 2 * p

            @pl.when(p > 0)
            def _():
                wait_store(buf0, s0)

            ga = fire(ka, buf0, g0)
            for cp in ga:
                cp.wait()
            pltpu.async_copy(buf0, out_hbm.at[pl.ds(base + ka * NB, NB)], s0)

            @pl.when(p > 0)
            def _():
                wait_store(buf1, s1)

            gb = fire(ka + 1, buf1, g1)
            for cp in gb:
                cp.wait()
            pltpu.async_copy(
                buf1, out_hbm.at[pl.ds(base + (ka + 1) * NB, NB)], s1
            )
            return carry

        lax.fori_loop(0, npairs, pair, 0)

        wait_store(buf0, s0)
        wait_store(buf1, s1)

    return gather


def kernel(x, weight):
    return _make_gather(x.shape[0], x.shape[1])(x, weight)
